# Initial kernel scaffold; baseline (speedup 1.0000x reference)
import functools

import jax
import jax.numpy as jnp
from jax import lax
from jax.experimental import pallas as pl
from jax.experimental.pallas import tpu as pltpu
from jax.experimental.pallas import tpu_sc as plsc

NC = 2
NS = 16
NW = NC * NS
C = 80


def _sc_degrees(np_, nch, src3, dst3, ones_c, zeros1):
    rpt = np_ // NS

    mesh = plsc.VectorSubcoreMesh(core_axis_name="c", subcore_axis_name="s")

    @functools.partial(
        pl.kernel,
        out_type=(
            jax.ShapeDtypeStruct((NC, np_), jnp.float32),
            jax.ShapeDtypeStruct((NC, np_), jnp.float32),
        ),
        mesh=mesh,
        scratch_types=[
            pltpu.VMEM((nch, C), jnp.int32),
            pltpu.VMEM((nch, C), jnp.int32),
            pltpu.VMEM((C,), jnp.float32),
            pltpu.VMEM_SHARED((np_,), jnp.float32),
            pltpu.VMEM_SHARED((np_,), jnp.float32),
        ],
    )
    def deg_kernel(src_hbm, dst_hbm, ones_hbm, zeros_hbm, dego_hbm, degi_hbm,
                   srcv, dstv, onesv, dego_sh, degi_sh):
        cid = lax.axis_index("c")
        sid = lax.axis_index("s")
        wid = cid * NS + sid
        pltpu.sync_copy(zeros_hbm, dego_sh.at[pl.ds(sid * rpt, rpt)])
        pltpu.sync_copy(zeros_hbm, degi_sh.at[pl.ds(sid * rpt, rpt)])
        pltpu.sync_copy(src_hbm.at[wid], srcv)
        pltpu.sync_copy(dst_hbm.at[wid], dstv)
        pltpu.sync_copy(ones_hbm, onesv)
        plsc.subcore_barrier()

        def body(c, carry):
            pltpu.sync_copy(onesv, dego_sh.at[srcv.at[c]], add=True)
            pltpu.sync_copy(onesv, degi_sh.at[dstv.at[c]], add=True)
            return carry

        lax.fori_loop(0, nch, body, 0)
        plsc.subcore_barrier()
        sl = pl.ds(sid * rpt, rpt)
        pltpu.sync_copy(dego_sh.at[sl], dego_hbm.at[cid, sl])
        pltpu.sync_copy(degi_sh.at[sl], degi_hbm.at[cid, sl])

    return deg_kernel(src3, dst3, ones_c, zeros1)


def _sc_spmm(np_, nch, xs, norm_in, src3, dst3, zeros1, zeros2):
    d = xs.shape[1]
    rpt = np_ // NS

    mesh = plsc.VectorSubcoreMesh(core_axis_name="c", subcore_axis_name="s")

    @functools.partial(
        pl.kernel,
        out_type=(
            jax.ShapeDtypeStruct((NC, np_, d), jnp.float32),
            jax.ShapeDtypeStruct((NC, np_), jnp.float32),
        ),
        mesh=mesh,
        scratch_types=[
            pltpu.VMEM((nch, C), jnp.int32),
            pltpu.VMEM((nch, C), jnp.int32),
            pltpu.VMEM((C, d), jnp.float32),
            pltpu.VMEM((C,), jnp.float32),
            pltpu.VMEM_SHARED((np_, d), jnp.float32),
            pltpu.VMEM_SHARED((np_,), jnp.float32),
            pltpu.SemaphoreType.DMA,
        ],
    )
    def spmm_kernel(xs_hbm, nin_hbm, src_hbm, dst_hbm, zeros1_hbm, zeros2_hbm,
                    agg_hbm, w_hbm, srcv, dstv, rows, nvals, agg_sh, w_sh, sem):
        cid = lax.axis_index("c")
        sid = lax.axis_index("s")
        wid = cid * NS + sid
        sl = pl.ds(sid * rpt, rpt)
        pltpu.sync_copy(zeros2_hbm, agg_sh.at[sl])
        pltpu.sync_copy(zeros1_hbm, w_sh.at[sl])
        pltpu.sync_copy(src_hbm.at[wid], srcv)
        pltpu.sync_copy(dst_hbm.at[wid], dstv)
        plsc.subcore_barrier()

        def body(c, carry):
            si = srcv.at[c]
            di = dstv.at[c]
            pltpu.async_copy(xs_hbm.at[si], rows, sem).wait()
            pltpu.sync_copy(rows, agg_sh.at[di], add=True)
            pltpu.async_copy(nin_hbm.at[di], nvals, sem).wait()
            pltpu.sync_copy(nvals, w_sh.at[si], add=True)
            return carry

        lax.fori_loop(0, nch, body, 0)
        plsc.subcore_barrier()
        pltpu.sync_copy(agg_sh.at[sl], agg_hbm.at[cid, sl])
        pltpu.sync_copy(w_sh.at[sl], w_hbm.at[cid, sl])

    return spmm_kernel(xs, norm_in, src3, dst3, zeros1, zeros2)


def _tc_norms(dop_t, dip_t, x_pad):
    np_, d = x_pad.shape

    def body(dop_ref, dip_ref, x_ref, xs_ref, no_ref, ni_ref):
        dego = jnp.sum(dop_ref[...], axis=1, keepdims=True)
        degi = jnp.sum(dip_ref[...], axis=1, keepdims=True)
        no = lax.rsqrt(jnp.maximum(dego, 1.0))
        ni = lax.rsqrt(jnp.maximum(degi, 1.0))
        no_ref[...] = no
        ni_ref[...] = ni
        xs_ref[...] = x_ref[...] * no

    return pl.pallas_call(
        body,
        out_shape=(
            jax.ShapeDtypeStruct((np_, d), jnp.float32),
            jax.ShapeDtypeStruct((np_, 1), jnp.float32),
            jax.ShapeDtypeStruct((np_, 1), jnp.float32),
        ),
    )(dop_t, dip_t, x_pad)


def _tc_final(n_real, agg0, agg1, w0, w1, ni, no, W1, b1, W2, b2, Wl, bl):
    np_, d = agg0.shape

    def body(agg0_ref, agg1_ref, w0_ref, w1_ref, ni_ref, no_ref,
             W1_ref, b1_ref, W2_ref, b2_ref, Wl_ref, bl_ref, out_ref):
        agg = agg0_ref[...] + agg1_ref[...]
        t = agg * ni_ref[...]
        h1 = jnp.dot(t, W1_ref[...], preferred_element_type=jnp.float32)
        h1 = jnp.maximum(h1 + b1_ref[...], 0.0)
        w = (w0_ref[...] + w1_ref[...]) * no_ref[...]
        mask = lax.broadcasted_iota(jnp.int32, (np_, 1), 0) < n_real
        w = jnp.where(mask, w, 0.0)
        v = jnp.sum(h1 * w, axis=0, keepdims=True)
        readout = jnp.dot(v, W2_ref[...], preferred_element_type=jnp.float32)
        readout = readout * (1.0 / n_real) + b2_ref[...]
        out = jnp.dot(readout, Wl_ref[...], preferred_element_type=jnp.float32)
        out_ref[...] = out + bl_ref[...]

    return pl.pallas_call(
        body,
        out_shape=jax.ShapeDtypeStruct((1, bl.shape[-1]), jnp.float32),
    )(agg0, agg1, w0, w1, ni, no, W1, b1, W2, b2, Wl, bl)


def kernel(x, edge_index, W1, b1, W2, b2, Wl, bl):
    n, d = x.shape
    e = edge_index.shape[1]

    np_ = ((n + 127) // 128) * 128
    pad_node = np_ - 1

    epw = ((e + NW * C - 1) // (NW * C)) * C
    ep = epw * NW
    nch = epw // C

    src = edge_index[0].astype(jnp.int32)
    dst = edge_index[1].astype(jnp.int32)
    if ep != e:
        fill = jnp.full((ep - e,), pad_node, dtype=jnp.int32)
        src = jnp.concatenate([src, fill])
        dst = jnp.concatenate([dst, fill])
    src3 = src.reshape(NW, nch, C)
    dst3 = dst.reshape(NW, nch, C)

    x_pad = jnp.zeros((np_, d), jnp.float32).at[:n].set(x)

    rpt = np_ // NS
    ones_c = jnp.ones((C,), jnp.float32)
    zeros1 = jnp.zeros((rpt,), jnp.float32)
    zeros2 = jnp.zeros((rpt, d), jnp.float32)

    deg_out_p, deg_in_p = _sc_degrees(np_, nch, src3, dst3, ones_c, zeros1)

    xs, norm_out, norm_in = _tc_norms(deg_out_p.T, deg_in_p.T, x_pad)

    agg_p, w_p = _sc_spmm(np_, nch, xs, norm_in.reshape(np_), src3, dst3,
                          zeros1, zeros2)

    out = _tc_final(n, agg_p[0], agg_p[1], w_p[0][:, None], w_p[1][:, None],
                    norm_in, norm_out, W1, b1[None, :], W2, b2[None, :],
                    Wl, bl[None, :])
    return out


# speedup vs baseline: 7.0085x; 7.0085x over previous
import functools

import jax
import jax.numpy as jnp
from jax import lax
from jax.experimental import pallas as pl
from jax.experimental.pallas import tpu as pltpu
from jax.experimental.pallas import tpu_sc as plsc

NC = 2
NS = 16
NW = NC * NS
C = 128


def _sc_degrees(np_, nch, src3, dst3, ones_c, zeros1):
    rpt = np_ // NS

    mesh = plsc.VectorSubcoreMesh(core_axis_name="c", subcore_axis_name="s")

    @functools.partial(
        pl.kernel,
        out_type=(
            jax.ShapeDtypeStruct((NC * np_,), jnp.float32),
            jax.ShapeDtypeStruct((NC * np_,), jnp.float32),
        ),
        mesh=mesh,
        scratch_types=[
            pltpu.VMEM((nch, C), jnp.int32),
            pltpu.VMEM((nch, C), jnp.int32),
            pltpu.VMEM((C,), jnp.float32),
            pltpu.VMEM((rpt,), jnp.float32),
            pltpu.VMEM_SHARED((np_,), jnp.float32),
            pltpu.VMEM_SHARED((np_,), jnp.float32),
        ],
    )
    def deg_kernel(src_hbm, dst_hbm, ones_hbm, zeros_hbm, dego_hbm, degi_hbm,
                   srcv, dstv, onesv, zbuf, dego_sh, degi_sh):
        cid = lax.axis_index("c")
        sid = lax.axis_index("s")
        wid = cid * NS + sid
        sl = pl.ds(sid * rpt, rpt)
        pltpu.sync_copy(zeros_hbm, zbuf)
        pltpu.sync_copy(zbuf, dego_sh.at[sl])
        pltpu.sync_copy(zbuf, degi_sh.at[sl])
        pltpu.sync_copy(src_hbm.at[wid], srcv)
        pltpu.sync_copy(dst_hbm.at[wid], dstv)
        pltpu.sync_copy(ones_hbm, onesv)
        plsc.subcore_barrier()

        def body(c, carry):
            pltpu.sync_copy(onesv, dego_sh.at[srcv.at[c]], add=True)
            pltpu.sync_copy(onesv, degi_sh.at[dstv.at[c]], add=True)
            return carry

        lax.fori_loop(0, nch, body, 0)
        plsc.subcore_barrier()
        osl = pl.ds(cid * np_ + sid * rpt, rpt)
        pltpu.sync_copy(dego_sh.at[sl], zbuf)
        pltpu.sync_copy(zbuf, dego_hbm.at[osl])
        pltpu.sync_copy(degi_sh.at[sl], zbuf)
        pltpu.sync_copy(zbuf, degi_hbm.at[osl])

    dego, degi = deg_kernel(src3, dst3, ones_c, zeros1)
    return dego.reshape(NC, np_), degi.reshape(NC, np_)


def _sc_spmm(np_, nch, xs, norm_in, src3, dst3, zeros1, zeros2):
    d = xs.shape[1]
    rpt = np_ // NS
    nzc = rpt // C

    mesh = plsc.VectorSubcoreMesh(core_axis_name="c", subcore_axis_name="s")

    @functools.partial(
        pl.kernel,
        out_type=(
            jax.ShapeDtypeStruct((NC, np_, d), jnp.float32),
            jax.ShapeDtypeStruct((NC * np_,), jnp.float32),
        ),
        mesh=mesh,
        scratch_types=[
            pltpu.VMEM((nch, C), jnp.int32),
            pltpu.VMEM((nch, C), jnp.int32),
            pltpu.VMEM((C, d), jnp.float32),
            pltpu.VMEM((C,), jnp.float32),
            pltpu.VMEM((rpt,), jnp.float32),
            pltpu.VMEM_SHARED((np_, d), jnp.float32),
            pltpu.VMEM_SHARED((np_,), jnp.float32),
            pltpu.SemaphoreType.DMA,
        ],
    )
    def spmm_kernel(xs_hbm, nin_hbm, src_hbm, dst_hbm, zeros1_hbm, zeros2_hbm,
                    agg_hbm, w_hbm, srcv, dstv, rows, nvals, zbuf,
                    agg_sh, w_sh, sem):
        cid = lax.axis_index("c")
        sid = lax.axis_index("s")
        wid = cid * NS + sid
        sl = pl.ds(sid * rpt, rpt)
        pltpu.sync_copy(zeros2_hbm, rows)
        for k in range(nzc):
            pltpu.sync_copy(rows, agg_sh.at[pl.ds(sid * rpt + k * C, C)])
        pltpu.sync_copy(zeros1_hbm, zbuf)
        pltpu.sync_copy(zbuf, w_sh.at[sl])
        pltpu.sync_copy(src_hbm.at[wid], srcv)
        pltpu.sync_copy(dst_hbm.at[wid], dstv)
        plsc.subcore_barrier()

        def body(c, carry):
            si = srcv.at[c]
            di = dstv.at[c]
            pltpu.async_copy(xs_hbm.at[si], rows, sem).wait()
            pltpu.sync_copy(rows, agg_sh.at[di], add=True)
            pltpu.async_copy(nin_hbm.at[di], nvals, sem).wait()
            pltpu.sync_copy(nvals, w_sh.at[si], add=True)
            return carry

        lax.fori_loop(0, nch, body, 0)
        plsc.subcore_barrier()
        for k in range(nzc):
            csl = pl.ds(sid * rpt + k * C, C)
            pltpu.sync_copy(agg_sh.at[csl], rows)
            pltpu.sync_copy(rows, agg_hbm.at[cid, csl])
        pltpu.sync_copy(w_sh.at[sl], zbuf)
        pltpu.sync_copy(zbuf, w_hbm.at[pl.ds(cid * np_ + sid * rpt, rpt)])

    agg_p, w_p = spmm_kernel(xs, norm_in, src3, dst3, zeros1, zeros2)
    return agg_p, w_p.reshape(NC, np_)


def _tc_norms(dop_t, dip_t, x_pad):
    np_, d = x_pad.shape

    def body(dop_ref, dip_ref, x_ref, xs_ref, no_ref, ni_ref):
        dego = jnp.sum(dop_ref[...], axis=1, keepdims=True)
        degi = jnp.sum(dip_ref[...], axis=1, keepdims=True)
        no = lax.rsqrt(jnp.maximum(dego, 1.0))
        ni = lax.rsqrt(jnp.maximum(degi, 1.0))
        no_ref[...] = no
        ni_ref[...] = ni
        xs_ref[...] = x_ref[...] * no

    return pl.pallas_call(
        body,
        out_shape=(
            jax.ShapeDtypeStruct((np_, d), jnp.float32),
            jax.ShapeDtypeStruct((np_, 1), jnp.float32),
            jax.ShapeDtypeStruct((np_, 1), jnp.float32),
        ),
    )(dop_t, dip_t, x_pad)


def _tc_final(n_real, agg0, agg1, w0, w1, ni, no, W1, b1, W2, b2, Wl, bl):
    np_, d = agg0.shape

    def body(agg0_ref, agg1_ref, w0_ref, w1_ref, ni_ref, no_ref,
             W1_ref, b1_ref, W2_ref, b2_ref, Wl_ref, bl_ref, out_ref):
        agg = agg0_ref[...] + agg1_ref[...]
        t = agg * ni_ref[...]
        h1 = jnp.dot(t, W1_ref[...], preferred_element_type=jnp.float32)
        h1 = jnp.maximum(h1 + b1_ref[...], 0.0)
        w = (w0_ref[...] + w1_ref[...]) * no_ref[...]
        mask = lax.broadcasted_iota(jnp.int32, (np_, 1), 0) < n_real
        w = jnp.where(mask, w, 0.0)
        v = jnp.sum(h1 * w, axis=0, keepdims=True)
        readout = jnp.dot(v, W2_ref[...], preferred_element_type=jnp.float32)
        readout = readout * (1.0 / n_real) + b2_ref[...]
        out = jnp.dot(readout, Wl_ref[...], preferred_element_type=jnp.float32)
        out_ref[...] = out + bl_ref[...]

    return pl.pallas_call(
        body,
        out_shape=jax.ShapeDtypeStruct((1, bl.shape[-1]), jnp.float32),
    )(agg0, agg1, w0, w1, ni, no, W1, b1, W2, b2, Wl, bl)


def kernel(x, edge_index, W1, b1, W2, b2, Wl, bl):
    n, d = x.shape
    e = edge_index.shape[1]

    np_ = ((n + NS * 128 - 1) // (NS * 128)) * (NS * 128)
    pad_node = np_ - 1

    nch = (e + NW * C - 1) // (NW * C)
    ep = nch * C * NW

    src = edge_index[0].astype(jnp.int32)
    dst = edge_index[1].astype(jnp.int32)
    if ep != e:
        fill = jnp.full((ep - e,), pad_node, dtype=jnp.int32)
        src = jnp.concatenate([src, fill])
        dst = jnp.concatenate([dst, fill])
    src3 = src.reshape(NW, nch, C)
    dst3 = dst.reshape(NW, nch, C)

    x_pad = jnp.zeros((np_, d), jnp.float32).at[:n].set(x)

    rpt = np_ // NS
    ones_c = jnp.ones((C,), jnp.float32)
    zeros1 = jnp.zeros((rpt,), jnp.float32)
    zeros2 = jnp.zeros((C, d), jnp.float32)

    deg_out_p, deg_in_p = _sc_degrees(np_, nch, src3, dst3, ones_c, zeros1)

    xs, norm_out, norm_in = _tc_norms(deg_out_p.T, deg_in_p.T, x_pad)

    agg_p, w_p = _sc_spmm(np_, nch, xs, norm_in.reshape(np_), src3, dst3,
                          zeros1, zeros2)

    out = _tc_final(n, agg_p[0], agg_p[1], w_p[0][:, None], w_p[1][:, None],
                    norm_in, norm_out, W1, b1[None, :], W2, b2[None, :],
                    Wl, bl[None, :])
    return out
